# baseline (speedup 1.0000x reference)
import jax
import jax.numpy as jnp
from jax import lax
from jax.experimental import pallas as pl
from jax.experimental.pallas import tpu as pltpu
from jax.experimental.pallas import tpu_sc as plsc

NUM_FIELDS = 26
VOCAB = 100000
EMB = 16
BATCH = 16384

NC, NS, L = 2, 16, 16
NW = NC * NS
B_PER_W = BATCH // NW
NB = 64
NCHUNK = B_PER_W // NB
IDS_PER_CHUNK = NB * NUM_FIELDS
SLICES_PER_CHUNK = IDS_PER_CHUNK // L
GATHER_ROWS = 128
NGATHER = IDS_PER_CHUNK // GATHER_ROWS
OUT_PER_B = EMB * NUM_FIELDS
OUT_PER_CHUNK = NB * OUT_PER_B


def _body(table_hbm, ids_hbm, out_hbm, offs_v,
          ids_v0, ids_v1, idx0, idx1, rows0, rows1, outb0, outb1,
          semg0, semg1, semo0, semo1):
  wid = lax.axis_index("s") * NC + lax.axis_index("c")

  iota = lax.iota(jnp.int32, L)
  e26 = iota * NUM_FIELDS

  @pl.loop(0, SLICES_PER_CHUNK, unroll=4)
  def _(s):
    p = iota + s * L
    f = lax.rem(p, NUM_FIELDS)
    offs_v[pl.ds(s * L, L)] = f * VOCAB

  bufs = [(ids_v0, idx0, rows0, outb0, semg0, semo0),
          (ids_v1, idx1, rows1, outb1, semg1, semo1)]

  def load_and_fire(c):
    ids_b, idx_b, rows_b, _, semg, _ = bufs[c % 2]
    ids_off = pl.multiple_of(
        wid * (B_PER_W * NUM_FIELDS) + c * IDS_PER_CHUNK, 8)
    pltpu.sync_copy(ids_hbm.at[pl.ds(ids_off, IDS_PER_CHUNK)], ids_b)

    @pl.loop(0, SLICES_PER_CHUNK, unroll=8)
    def _(s):
      flat = ids_b[pl.ds(s * L, L)] + offs_v[pl.ds(s * L, L)]
      r = lax.div(s, jnp.int32(GATHER_ROWS // L))
      k = lax.rem(s, jnp.int32(GATHER_ROWS // L))
      idx_b[r, pl.ds(k * L, L)] = flat

    return [
        pltpu.async_copy(table_hbm.at[idx_b.at[i]],
                         rows_b.at[pl.ds(i * GATHER_ROWS, GATHER_ROWS)],
                         semg)
        for i in range(NGATHER)
    ]

  def transpose(c):
    _, _, rows_b, out_b, _, _ = bufs[c % 2]

    @pl.loop(0, NUM_FIELDS)
    def _(j):
      @pl.loop(0, NB, unroll=8)
      def _(b, j=j):
        v = rows_b[b * NUM_FIELDS + j, :]
        plsc.store_scatter(out_b, [b * OUT_PER_B + j + e26], v)

  gath = {0: load_and_fire(0)}
  outcp = {}
  for c in range(NCHUNK):
    if c + 1 < NCHUNK:
      gath[c + 1] = load_and_fire(c + 1)
    for cp in gath.pop(c):
      cp.wait()
    if c - 2 in outcp:
      outcp.pop(c - 2).wait()
    transpose(c)
    out_b, semo = bufs[c % 2][3], bufs[c % 2][5]
    out_off = pl.multiple_of(
        wid * (B_PER_W * OUT_PER_B) + c * OUT_PER_CHUNK, 8)
    outcp[c] = pltpu.async_copy(
        out_b, out_hbm.at[pl.ds(out_off, OUT_PER_CHUNK)], semo)
  for c in sorted(outcp):
    outcp[c].wait()


@jax.jit
def _run(ids_flat, table):
  mesh = plsc.VectorSubcoreMesh(core_axis_name="c", subcore_axis_name="s",
                                num_cores=NC, num_subcores=NS)
  return pl.kernel(
      _body,
      out_type=jax.ShapeDtypeStruct((BATCH * OUT_PER_B,), jnp.float32),
      mesh=mesh,
      compiler_params=pltpu.CompilerParams(needs_layout_passes=False,
                                           use_tc_tiling_on_sc=False),
      scratch_types=[
          pltpu.VMEM((IDS_PER_CHUNK,), jnp.int32),
          pltpu.VMEM((IDS_PER_CHUNK,), jnp.int32),
          pltpu.VMEM((IDS_PER_CHUNK,), jnp.int32),
          pltpu.VMEM((NGATHER, GATHER_ROWS), jnp.int32),
          pltpu.VMEM((NGATHER, GATHER_ROWS), jnp.int32),
          pltpu.VMEM((IDS_PER_CHUNK, EMB), jnp.float32),
          pltpu.VMEM((IDS_PER_CHUNK, EMB), jnp.float32),
          pltpu.VMEM((OUT_PER_CHUNK,), jnp.float32),
          pltpu.VMEM((OUT_PER_CHUNK,), jnp.float32),
          pltpu.SemaphoreType.DMA,
          pltpu.SemaphoreType.DMA,
          pltpu.SemaphoreType.DMA,
          pltpu.SemaphoreType.DMA,
      ],
  )(table, ids_flat)


def kernel(ids, W):
  ids_flat = ids.astype(jnp.int32).reshape(BATCH * NUM_FIELDS)
  table = W.reshape(NUM_FIELDS * VOCAB, EMB)
  out = _run(ids_flat, table)
  return out.reshape(BATCH, EMB, NUM_FIELDS)


# speedup vs baseline: 1.1888x; 1.1888x over previous
import jax
import jax.numpy as jnp
from jax import lax
from jax.experimental import pallas as pl
from jax.experimental.pallas import tpu as pltpu
from jax.experimental.pallas import tpu_sc as plsc

NUM_FIELDS = 26
VOCAB = 100000
EMB = 16
BATCH = 16384

NC, NS, L = 2, 16, 16
NW = NC * NS
B_PER_W = BATCH // NW
NB = 64
NSUB = B_PER_W // NB
IDS_PER_SUB = NB * NUM_FIELDS
SLICES_PER_SUB = IDS_PER_SUB // L
GATHER_ROWS = 128
NGATHER = IDS_PER_SUB // GATHER_ROWS
TILE_W = 128
STAGE_PER_F = EMB * TILE_W
NBLK = B_PER_W // TILE_W


def _body(table_hbm, ids_hbm, out_hbm, offs_v,
          ids_v0, ids_v1, idx0, idx1, rows0, rows1, stage,
          semg0, semg1, semo):
  wid = lax.axis_index("s") * NC + lax.axis_index("c")

  iota = lax.iota(jnp.int32, L)
  g_vec = lax.div(iota, jnp.int32(8))
  r128 = lax.rem(iota, jnp.int32(8)) * TILE_W

  @pl.loop(0, SLICES_PER_SUB, unroll=4)
  def _(s):
    p = iota + s * L
    f = lax.rem(p, NUM_FIELDS)
    offs_v[pl.ds(s * L, L)] = f * VOCAB

  bufs = [(ids_v0, idx0, rows0, semg0), (ids_v1, idx1, rows1, semg1)]

  def load_and_fire(sc):
    ids_b, idx_b, rows_b, semg = bufs[sc % 2]
    ids_off = pl.multiple_of(
        wid * (B_PER_W * NUM_FIELDS) + sc * IDS_PER_SUB, 8)
    pltpu.sync_copy(ids_hbm.at[pl.ds(ids_off, IDS_PER_SUB)], ids_b)

    @pl.loop(0, SLICES_PER_SUB, unroll=8)
    def _(s):
      flat = ids_b[pl.ds(s * L, L)] + offs_v[pl.ds(s * L, L)]
      r = lax.div(s, jnp.int32(GATHER_ROWS // L))
      k = lax.rem(s, jnp.int32(GATHER_ROWS // L))
      idx_b[r, pl.ds(k * L, L)] = flat

    return [
        pltpu.async_copy(table_hbm.at[idx_b.at[i]],
                         rows_b.at[pl.ds(i * GATHER_ROWS, GATHER_ROWS)],
                         semg)
        for i in range(NGATHER)
    ]

  def scatter(sc):
    half = sc % 2
    rows_b = bufs[half][2]
    zero = iota * 0

    @pl.loop(0, NUM_FIELDS)
    def _(j):
      d0 = j * 2 + g_vec

      @pl.loop(0, NB, unroll=8)
      def _(b, j=j, d0=d0):
        v = rows_b[b * NUM_FIELDS + j, :]
        plsc.store_scatter(stage, [d0, zero, r128 + (half * NB + b)], v)

  def store_block(blk):
    bt = wid * NBLK + blk
    return pltpu.async_copy(stage, out_hbm.at[:, pl.ds(bt, 1), :], semo)

  gath = {0: load_and_fire(0)}
  outcp = None
  for sc in range(NSUB):
    if sc + 1 < NSUB:
      gath[sc + 1] = load_and_fire(sc + 1)
    for cp in gath.pop(sc):
      cp.wait()
    if sc % 2 == 0 and outcp is not None:
      outcp.wait()
      outcp = None
    scatter(sc)
    if sc % 2 == 1:
      outcp = store_block(sc // 2)
  if outcp is not None:
    outcp.wait()


@jax.jit
def _run(ids_flat, table):
  mesh = plsc.VectorSubcoreMesh(core_axis_name="c", subcore_axis_name="s",
                                num_cores=NC, num_subcores=NS)
  return pl.kernel(
      _body,
      out_type=jax.ShapeDtypeStruct((NUM_FIELDS * 2, BATCH // TILE_W, 1024),
                                    jnp.float32),
      mesh=mesh,
      compiler_params=pltpu.CompilerParams(needs_layout_passes=False,
                                           use_tc_tiling_on_sc=False),
      scratch_types=[
          pltpu.VMEM((IDS_PER_SUB,), jnp.int32),
          pltpu.VMEM((IDS_PER_SUB,), jnp.int32),
          pltpu.VMEM((IDS_PER_SUB,), jnp.int32),
          pltpu.VMEM((NGATHER, GATHER_ROWS), jnp.int32),
          pltpu.VMEM((NGATHER, GATHER_ROWS), jnp.int32),
          pltpu.VMEM((IDS_PER_SUB, EMB), jnp.float32),
          pltpu.VMEM((IDS_PER_SUB, EMB), jnp.float32),
          pltpu.VMEM((NUM_FIELDS * 2, 1, 1024), jnp.float32),
          pltpu.SemaphoreType.DMA,
          pltpu.SemaphoreType.DMA,
          pltpu.SemaphoreType.DMA,
      ],
  )(table, ids_flat)


def kernel(ids, W):
  ids_flat = ids.astype(jnp.int32).reshape(BATCH * NUM_FIELDS)
  table = W.reshape(NUM_FIELDS * VOCAB, EMB)
  out = _run(ids_flat, table)
  z = (out.reshape(NUM_FIELDS, 2, BATCH // TILE_W, 8, TILE_W)
       .transpose(2, 4, 1, 3, 0)
       .reshape(BATCH, EMB, NUM_FIELDS))
  return z
